# initial kernel scaffold (unmeasured)
import jax
import jax.numpy as jnp
from jax import lax
from jax.experimental import pallas as pl
from jax.experimental.pallas import tpu as pltpu

N_DEV = 4
M, K, N = 4096, 4096, 2048
CH = M // N_DEV


def _ring_allreduce(partial):
    m, n = partial.shape

    def body(p_ref, out_ref, comm_ref, rs_send, rs_recv, ag_send, ag_recv):
        my = lax.axis_index("i")
        left = (my - 1) % N_DEV
        right = (my + 1) % N_DEV

        barrier = pltpu.get_barrier_semaphore()
        for nbr in (left, right):
            pl.semaphore_signal(
                barrier, inc=1,
                device_id=(nbr,), device_id_type=pl.DeviceIdType.MESH,
            )
        pl.semaphore_wait(barrier, 2)

        out_ref[...] = p_ref[...]

        for s in range(N_DEV - 1):
            send_c = (my - s) % N_DEV
            recv_c = (my - s - 1) % N_DEV
            rdma = pltpu.make_async_remote_copy(
                src_ref=out_ref.at[pl.ds(send_c * CH, CH), :],
                dst_ref=comm_ref.at[s],
                send_sem=rs_send.at[s],
                recv_sem=rs_recv.at[s],
                device_id=(right,),
                device_id_type=pl.DeviceIdType.MESH,
            )
            rdma.start()
            rdma.wait()
            out_ref[pl.ds(recv_c * CH, CH), :] = (
                out_ref[pl.ds(recv_c * CH, CH), :] + comm_ref[s]
            )

        for s in range(N_DEV - 1):
            send_c = (my + 1 - s) % N_DEV
            rdma = pltpu.make_async_remote_copy(
                src_ref=out_ref.at[pl.ds(send_c * CH, CH), :],
                dst_ref=out_ref.at[pl.ds(send_c * CH, CH), :],
                send_sem=ag_send.at[s],
                recv_sem=ag_recv.at[s],
                device_id=(right,),
                device_id_type=pl.DeviceIdType.MESH,
            )
            rdma.start()
            rdma.wait()

    return pl.pallas_call(
        body,
        out_shape=jax.ShapeDtypeStruct((m, n), jnp.float32),
        in_specs=[pl.BlockSpec(memory_space=pltpu.VMEM)],
        out_specs=pl.BlockSpec(memory_space=pltpu.VMEM),
        scratch_shapes=[
            pltpu.VMEM((N_DEV - 1, CH, n), jnp.float32),
            pltpu.SemaphoreType.DMA((N_DEV - 1,)),
            pltpu.SemaphoreType.DMA((N_DEV - 1,)),
            pltpu.SemaphoreType.DMA((N_DEV - 1,)),
            pltpu.SemaphoreType.DMA((N_DEV - 1,)),
        ],
        compiler_params=pltpu.CompilerParams(collective_id=0),
    )(partial)


def kernel(x, w_mat):
    partial = jnp.dot(
        x, w_mat,
        precision=lax.Precision.HIGHEST,
        preferred_element_type=jnp.float32,
    )
    y = _ring_allreduce(partial)
    amax = jnp.max(jnp.abs(y))
    scale = amax / 127.0
    q = jnp.clip(jnp.round(y / scale), -127.0, 127.0)
    return (q * scale).astype(jnp.float32)


# baseline (device time: 729567 ns/iter reference)
import jax
import jax.numpy as jnp
from jax import lax
from jax.experimental import pallas as pl
from jax.experimental.pallas import tpu as pltpu

N_DEV = 4
M, K, N = 4096, 4096, 2048
CH = M // N_DEV


def _ring_allreduce(partial):
    m, n = partial.shape

    def body(p_hbm, out_hbm, own_ref, acc_ref, comm_ref,
             rs_send, rs_recv, ag_send, ag_recv, local_sem):
        my = lax.axis_index("i")
        left = (my - 1) % N_DEV
        right = (my + 1) % N_DEV

        barrier = pltpu.get_barrier_semaphore()
        for nbr in (left, right):
            pl.semaphore_signal(
                barrier, inc=1,
                device_id=(nbr,), device_id_type=pl.DeviceIdType.MESH,
            )
        pl.semaphore_wait(barrier, 2)

        cp = pltpu.make_async_copy(
            p_hbm.at[pl.ds(my * CH, CH), :], acc_ref, local_sem)
        cp.start()
        cp.wait()

        for s in range(N_DEV - 1):
            recv_c = (my - s - 1) % N_DEV
            rdma = pltpu.make_async_remote_copy(
                src_ref=acc_ref,
                dst_ref=comm_ref.at[s],
                send_sem=rs_send.at[s],
                recv_sem=rs_recv.at[s],
                device_id=(right,),
                device_id_type=pl.DeviceIdType.MESH,
            )
            rdma.start()
            cp = pltpu.make_async_copy(
                p_hbm.at[pl.ds(recv_c * CH, CH), :], own_ref, local_sem)
            cp.start()
            cp.wait()
            rdma.wait()
            acc_ref[...] = own_ref[...] + comm_ref[s]

        g0 = (my + 1) % N_DEV
        cp = pltpu.make_async_copy(
            acc_ref, out_hbm.at[pl.ds(g0 * CH, CH), :], local_sem)
        cp.start()
        cp.wait()

        for s in range(N_DEV - 1):
            send_c = (my + 1 - s) % N_DEV
            rdma = pltpu.make_async_remote_copy(
                src_ref=out_hbm.at[pl.ds(send_c * CH, CH), :],
                dst_ref=out_hbm.at[pl.ds(send_c * CH, CH), :],
                send_sem=ag_send.at[s],
                recv_sem=ag_recv.at[s],
                device_id=(right,),
                device_id_type=pl.DeviceIdType.MESH,
            )
            rdma.start()
            rdma.wait()

    return pl.pallas_call(
        body,
        out_shape=jax.ShapeDtypeStruct((m, n), jnp.float32),
        in_specs=[pl.BlockSpec(memory_space=pl.ANY)],
        out_specs=pl.BlockSpec(memory_space=pl.ANY),
        scratch_shapes=[
            pltpu.VMEM((CH, n), jnp.float32),
            pltpu.VMEM((CH, n), jnp.float32),
            pltpu.VMEM((N_DEV - 1, CH, n), jnp.float32),
            pltpu.SemaphoreType.DMA((N_DEV - 1,)),
            pltpu.SemaphoreType.DMA((N_DEV - 1,)),
            pltpu.SemaphoreType.DMA((N_DEV - 1,)),
            pltpu.SemaphoreType.DMA((N_DEV - 1,)),
            pltpu.SemaphoreType.DMA,
        ],
        compiler_params=pltpu.CompilerParams(
            collective_id=0,
            vmem_limit_bytes=63 * 1024 * 1024,
        ),
    )(partial)


def kernel(x, w_mat):
    partial = jnp.dot(
        x, w_mat,
        precision=lax.Precision.HIGHEST,
        preferred_element_type=jnp.float32,
    )
    y = _ring_allreduce(partial)
    amax = jnp.max(jnp.abs(y))
    scale = amax / 127.0
    q = jnp.clip(jnp.round(y / scale), -127.0, 127.0)
    return (q * scale).astype(jnp.float32)


# device time: 251228 ns/iter; 2.9040x vs baseline; 2.9040x over previous
import jax
import jax.numpy as jnp
from jax import lax
from jax.experimental import pallas as pl
from jax.experimental.pallas import tpu as pltpu

N_DEV = 4
M, K, N = 4096, 4096, 2048
CH = M // N_DEV
H = N // 2


def _allreduce_quant(partial):
    def body(p_hbm, out_hbm,
             acc0, acc1, own0, own1, comm0, comm1, q0, q1, ag0, ag1, axb,
             rs0_send, rs0_recv, rs1_send, rs1_recv,
             ag0_send, ag0_recv, ag1_send, ag1_recv,
             ax_send, ax_recv, l0, l1, lo0, lo1):
        my = lax.axis_index("i")
        left = (my - 1) % N_DEV
        right = (my + 1) % N_DEV

        def remote(src, dst, ssem, rsem, tgt):
            return pltpu.make_async_remote_copy(
                src_ref=src, dst_ref=dst, send_sem=ssem, recv_sem=rsem,
                device_id=(tgt,), device_id_type=pl.DeviceIdType.MESH,
            )

        barrier = pltpu.get_barrier_semaphore()
        for nbr in (left, right):
            pl.semaphore_signal(
                barrier, inc=1,
                device_id=(nbr,), device_id_type=pl.DeviceIdType.MESH,
            )
        pl.semaphore_wait(barrier, 2)

        cp0 = pltpu.make_async_copy(
            p_hbm.at[pl.ds(my * CH, CH), pl.ds(0, H)], acc0, l0)
        cp1 = pltpu.make_async_copy(
            p_hbm.at[pl.ds(my * CH, CH), pl.ds(H, H)], acc1, l1)
        cp0.start()
        cp1.start()
        cp0.wait()
        cp1.wait()

        for s in range(N_DEV - 1):
            r0 = remote(acc0, comm0.at[s], rs0_send.at[s], rs0_recv.at[s],
                        right)
            r1 = remote(acc1, comm1.at[s], rs1_send.at[s], rs1_recv.at[s],
                        left)
            r0.start()
            r1.start()
            c0 = (my - s - 1) % N_DEV
            c1 = (my + s + 1) % N_DEV
            cp0 = pltpu.make_async_copy(
                p_hbm.at[pl.ds(c0 * CH, CH), pl.ds(0, H)], own0, l0)
            cp1 = pltpu.make_async_copy(
                p_hbm.at[pl.ds(c1 * CH, CH), pl.ds(H, H)], own1, l1)
            cp0.start()
            cp1.start()
            cp0.wait()
            cp1.wait()
            r0.wait()
            r1.wait()
            acc0[...] = own0[...] + comm0[s]
            acc1[...] = own1[...] + comm1[s]

        my_amax = jnp.maximum(jnp.max(jnp.abs(acc0[...])),
                              jnp.max(jnp.abs(acc1[...])))
        axb[0] = jnp.full((8, 128), my_amax, jnp.float32)
        a_r = remote(axb.at[0], axb.at[1], ax_send.at[0], ax_recv.at[0],
                     right)
        a_l = remote(axb.at[0], axb.at[2], ax_send.at[1], ax_recv.at[1],
                     left)
        a_r.start()
        a_l.start()
        a_r.wait()
        a_l.wait()
        a_f = remote(axb.at[1], axb.at[3], ax_send.at[2], ax_recv.at[2],
                     right)
        a_f.start()
        a_f.wait()
        amax = jnp.max(jnp.stack([axb[0], axb[1], axb[2], axb[3]]))
        scale = amax / 127.0
        inv = 127.0 / amax

        g0 = (my + 1) % N_DEV
        g1 = (my - 1) % N_DEV
        q0[...] = jnp.clip(jnp.round(acc0[...] * inv), -127.0, 127.0
                           ).astype(jnp.int8)
        q1[...] = jnp.clip(jnp.round(acc1[...] * inv), -127.0, 127.0
                           ).astype(jnp.int8)
        own0[...] = q0[...].astype(jnp.float32) * scale
        own1[...] = q1[...].astype(jnp.float32) * scale
        st0 = pltpu.make_async_copy(
            own0, out_hbm.at[pl.ds(g0 * CH, CH), pl.ds(0, H)], lo0)
        st1 = pltpu.make_async_copy(
            own1, out_hbm.at[pl.ds(g1 * CH, CH), pl.ds(H, H)], lo1)
        st0.start()
        st1.start()

        for s in range(N_DEV - 1):
            src0 = q0 if s == 0 else ag0.at[s - 1]
            src1 = q1 if s == 0 else ag1.at[s - 1]
            r0 = remote(src0, ag0.at[s], ag0_send.at[s], ag0_recv.at[s],
                        right)
            r1 = remote(src1, ag1.at[s], ag1_send.at[s], ag1_recv.at[s],
                        left)
            r0.start()
            r1.start()
            r0.wait()
            r1.wait()
            st0.wait()
            st1.wait()
            a0 = (my - s) % N_DEV
            a1 = (my + s) % N_DEV
            own0[...] = ag0[s].astype(jnp.float32) * scale
            own1[...] = ag1[s].astype(jnp.float32) * scale
            st0 = pltpu.make_async_copy(
                own0, out_hbm.at[pl.ds(a0 * CH, CH), pl.ds(0, H)], lo0)
            st1 = pltpu.make_async_copy(
                own1, out_hbm.at[pl.ds(a1 * CH, CH), pl.ds(H, H)], lo1)
            st0.start()
            st1.start()
        st0.wait()
        st1.wait()

    return pl.pallas_call(
        body,
        out_shape=jax.ShapeDtypeStruct((M, N), jnp.float32),
        in_specs=[pl.BlockSpec(memory_space=pl.ANY)],
        out_specs=pl.BlockSpec(memory_space=pl.ANY),
        scratch_shapes=[
            pltpu.VMEM((CH, H), jnp.float32),
            pltpu.VMEM((CH, H), jnp.float32),
            pltpu.VMEM((CH, H), jnp.float32),
            pltpu.VMEM((CH, H), jnp.float32),
            pltpu.VMEM((N_DEV - 1, CH, H), jnp.float32),
            pltpu.VMEM((N_DEV - 1, CH, H), jnp.float32),
            pltpu.VMEM((CH, H), jnp.int8),
            pltpu.VMEM((CH, H), jnp.int8),
            pltpu.VMEM((N_DEV - 1, CH, H), jnp.int8),
            pltpu.VMEM((N_DEV - 1, CH, H), jnp.int8),
            pltpu.VMEM((4, 8, 128), jnp.float32),
            pltpu.SemaphoreType.DMA((N_DEV - 1,)),
            pltpu.SemaphoreType.DMA((N_DEV - 1,)),
            pltpu.SemaphoreType.DMA((N_DEV - 1,)),
            pltpu.SemaphoreType.DMA((N_DEV - 1,)),
            pltpu.SemaphoreType.DMA((N_DEV - 1,)),
            pltpu.SemaphoreType.DMA((N_DEV - 1,)),
            pltpu.SemaphoreType.DMA((N_DEV - 1,)),
            pltpu.SemaphoreType.DMA((N_DEV - 1,)),
            pltpu.SemaphoreType.DMA((3,)),
            pltpu.SemaphoreType.DMA((3,)),
            pltpu.SemaphoreType.DMA,
            pltpu.SemaphoreType.DMA,
            pltpu.SemaphoreType.DMA,
            pltpu.SemaphoreType.DMA,
        ],
        compiler_params=pltpu.CompilerParams(
            collective_id=0,
            vmem_limit_bytes=63 * 1024 * 1024,
        ),
    )(partial)


def kernel(x, w_mat):
    partial = jnp.dot(x, w_mat, preferred_element_type=jnp.float32)
    return _allreduce_quant(partial)


# device time: 230947 ns/iter; 3.1590x vs baseline; 1.0878x over previous
import jax
import jax.numpy as jnp
from jax import lax
from jax.experimental import pallas as pl
from jax.experimental.pallas import tpu as pltpu

N_DEV = 4
M, K, N = 4096, 4096, 2048
CH = M // N_DEV
H = N // 2


def _fused_gemm_ar_quant(x, w_mat):
    kx = x.shape[1]

    def body(x_hbm, w_ref, out_hbm,
             xs0, xs1, acc0, acc1, dq0, dq1, comm0, comm1,
             q0, q1, ag0, ag1, axb,
             rs0_send, rs0_recv, rs1_send, rs1_recv,
             ag0_send, ag0_recv, ag1_send, ag1_recv,
             ax_send, ax_recv, lx0, lx1, lo0, lo1):
        my = lax.axis_index("i")
        left = (my - 1) % N_DEV
        right = (my + 1) % N_DEV

        def remote(src, dst, ssem, rsem, tgt):
            return pltpu.make_async_remote_copy(
                src_ref=src, dst_ref=dst, send_sem=ssem, recv_sem=rsem,
                device_id=(tgt,), device_id_type=pl.DeviceIdType.MESH,
            )

        barrier = pltpu.get_barrier_semaphore()
        for nbr in (left, right):
            pl.semaphore_signal(
                barrier, inc=1,
                device_id=(nbr,), device_id_type=pl.DeviceIdType.MESH,
            )
        pl.semaphore_wait(barrier, 2)

        cx = pltpu.make_async_copy(
            x_hbm.at[pl.ds(my * CH, CH), :], xs0, lx0)
        cx.start()
        cx.wait()
        acc0[...] = jnp.dot(xs0[...], w_ref[:, pl.ds(0, H)],
                            preferred_element_type=jnp.float32)
        acc1[...] = jnp.dot(xs0[...], w_ref[:, pl.ds(H, H)],
                            preferred_element_type=jnp.float32)

        for s in range(N_DEV - 1):
            r0 = remote(acc0, comm0.at[s % 2], rs0_send.at[s],
                        rs0_recv.at[s], right)
            r1 = remote(acc1, comm1.at[s % 2], rs1_send.at[s],
                        rs1_recv.at[s], left)
            r0.start()
            r1.start()
            c0 = (my - s - 1) % N_DEV
            c1 = (my + s + 1) % N_DEV
            cx0 = pltpu.make_async_copy(
                x_hbm.at[pl.ds(c0 * CH, CH), :], xs0, lx0)
            cx1 = pltpu.make_async_copy(
                x_hbm.at[pl.ds(c1 * CH, CH), :], xs1, lx1)
            cx0.start()
            cx1.start()
            cx0.wait()
            cx1.wait()
            dq0[...] = jnp.dot(xs0[...], w_ref[:, pl.ds(0, H)],
                               preferred_element_type=jnp.float32)
            dq1[...] = jnp.dot(xs1[...], w_ref[:, pl.ds(H, H)],
                               preferred_element_type=jnp.float32)
            r0.wait()
            r1.wait()
            acc0[...] = dq0[...] + comm0[s % 2]
            acc1[...] = dq1[...] + comm1[s % 2]

        my_amax = jnp.maximum(jnp.max(jnp.abs(acc0[...])),
                              jnp.max(jnp.abs(acc1[...])))
        axb[0] = jnp.full((8, 128), my_amax, jnp.float32)
        a_r = remote(axb.at[0], axb.at[1], ax_send.at[0], ax_recv.at[0],
                     right)
        a_l = remote(axb.at[0], axb.at[2], ax_send.at[1], ax_recv.at[1],
                     left)
        a_r.start()
        a_l.start()
        a_r.wait()
        a_l.wait()
        a_f = remote(axb.at[1], axb.at[3], ax_send.at[2], ax_recv.at[2],
                     right)
        a_f.start()
        a_f.wait()
        amax = jnp.max(jnp.stack([axb[0], axb[1], axb[2], axb[3]]))
        scale = amax / 127.0
        inv = 127.0 / amax

        g0 = (my + 1) % N_DEV
        g1 = (my - 1) % N_DEV
        q0[...] = jnp.clip(jnp.round(acc0[...] * inv), -127.0, 127.0
                           ).astype(jnp.int8)
        q1[...] = jnp.clip(jnp.round(acc1[...] * inv), -127.0, 127.0
                           ).astype(jnp.int8)
        dq0[...] = q0[...].astype(jnp.float32) * scale
        dq1[...] = q1[...].astype(jnp.float32) * scale
        st0 = pltpu.make_async_copy(
            dq0, out_hbm.at[pl.ds(g0 * CH, CH), pl.ds(0, H)], lo0)
        st1 = pltpu.make_async_copy(
            dq1, out_hbm.at[pl.ds(g1 * CH, CH), pl.ds(H, H)], lo1)
        st0.start()
        st1.start()

        for s in range(N_DEV - 1):
            src0 = q0 if s == 0 else ag0.at[s - 1]
            src1 = q1 if s == 0 else ag1.at[s - 1]
            r0 = remote(src0, ag0.at[s], ag0_send.at[s], ag0_recv.at[s],
                        right)
            r1 = remote(src1, ag1.at[s], ag1_send.at[s], ag1_recv.at[s],
                        left)
            r0.start()
            r1.start()
            r0.wait()
            r1.wait()
            st0.wait()
            st1.wait()
            a0 = (my - s) % N_DEV
            a1 = (my + s) % N_DEV
            dq0[...] = ag0[s].astype(jnp.float32) * scale
            dq1[...] = ag1[s].astype(jnp.float32) * scale
            st0 = pltpu.make_async_copy(
                dq0, out_hbm.at[pl.ds(a0 * CH, CH), pl.ds(0, H)], lo0)
            st1 = pltpu.make_async_copy(
                dq1, out_hbm.at[pl.ds(a1 * CH, CH), pl.ds(H, H)], lo1)
            st0.start()
            st1.start()
        st0.wait()
        st1.wait()

    return pl.pallas_call(
        body,
        out_shape=jax.ShapeDtypeStruct((M, N), jnp.float32),
        in_specs=[
            pl.BlockSpec(memory_space=pl.ANY),
            pl.BlockSpec(memory_space=pltpu.MemorySpace.VMEM),
        ],
        out_specs=pl.BlockSpec(memory_space=pl.ANY),
        scratch_shapes=[
            pltpu.VMEM((CH, K // N_DEV), jnp.float32),
            pltpu.VMEM((CH, K // N_DEV), jnp.float32),
            pltpu.VMEM((CH, H), jnp.float32),
            pltpu.VMEM((CH, H), jnp.float32),
            pltpu.VMEM((CH, H), jnp.float32),
            pltpu.VMEM((CH, H), jnp.float32),
            pltpu.VMEM((2, CH, H), jnp.float32),
            pltpu.VMEM((2, CH, H), jnp.float32),
            pltpu.VMEM((CH, H), jnp.int8),
            pltpu.VMEM((CH, H), jnp.int8),
            pltpu.VMEM((N_DEV - 1, CH, H), jnp.int8),
            pltpu.VMEM((N_DEV - 1, CH, H), jnp.int8),
            pltpu.VMEM((4, 8, 128), jnp.float32),
            pltpu.SemaphoreType.DMA((N_DEV - 1,)),
            pltpu.SemaphoreType.DMA((N_DEV - 1,)),
            pltpu.SemaphoreType.DMA((N_DEV - 1,)),
            pltpu.SemaphoreType.DMA((N_DEV - 1,)),
            pltpu.SemaphoreType.DMA((N_DEV - 1,)),
            pltpu.SemaphoreType.DMA((N_DEV - 1,)),
            pltpu.SemaphoreType.DMA((N_DEV - 1,)),
            pltpu.SemaphoreType.DMA((N_DEV - 1,)),
            pltpu.SemaphoreType.DMA((3,)),
            pltpu.SemaphoreType.DMA((3,)),
            pltpu.SemaphoreType.DMA,
            pltpu.SemaphoreType.DMA,
            pltpu.SemaphoreType.DMA,
            pltpu.SemaphoreType.DMA,
        ],
        compiler_params=pltpu.CompilerParams(
            collective_id=0,
            vmem_limit_bytes=63 * 1024 * 1024,
        ),
    )(x, w_mat)


def kernel(x, w_mat):
    return _fused_gemm_ar_quant(x, w_mat)


# device time: 162781 ns/iter; 4.4819x vs baseline; 1.4188x over previous
import jax
import jax.numpy as jnp
from jax import lax
from jax.experimental import pallas as pl
from jax.experimental.pallas import tpu as pltpu

N_DEV = 4
M, K, N = 4096, 4096, 2048
CH = M // N_DEV
H = N // 2


def _fused_gemm_ar_quant(x, w_mat):
    kx = x.shape[1]

    def body(x_hbm, w_ref, out_hbm,
             xs0, xs1, acc0, acc1, dq0, dq1, sb0, sb1, comm0, comm1,
             q0, q1, ag0, ag1, axb,
             rs0_send, rs0_recv, rs1_send, rs1_recv,
             ag0_send, ag0_recv, ag1_send, ag1_recv,
             ax_send, ax_recv, lx0, lx1, lo0, lo1):
        my = lax.axis_index("i")
        left = (my - 1) % N_DEV
        right = (my + 1) % N_DEV

        def remote(src, dst, ssem, rsem, tgt):
            return pltpu.make_async_remote_copy(
                src_ref=src, dst_ref=dst, send_sem=ssem, recv_sem=rsem,
                device_id=(tgt,), device_id_type=pl.DeviceIdType.MESH,
            )

        barrier = pltpu.get_barrier_semaphore()
        for nbr in (left, right):
            pl.semaphore_signal(
                barrier, inc=1,
                device_id=(nbr,), device_id_type=pl.DeviceIdType.MESH,
            )
        pl.semaphore_wait(barrier, 2)

        cx = pltpu.make_async_copy(
            x_hbm.at[pl.ds(my * CH, CH), :], xs0, lx0)
        cx.start()
        cx.wait()
        acc0[...] = jnp.dot(xs0[...], w_ref[:, pl.ds(0, H)],
                            preferred_element_type=jnp.float32)
        acc1[...] = jnp.dot(xs0[...], w_ref[:, pl.ds(H, H)],
                            preferred_element_type=jnp.float32)

        for s in range(N_DEV - 1):
            sb0[...] = acc0[...].astype(jnp.bfloat16)
            sb1[...] = acc1[...].astype(jnp.bfloat16)
            r0 = remote(sb0, comm0.at[s % 2], rs0_send.at[s],
                        rs0_recv.at[s], right)
            r1 = remote(sb1, comm1.at[s % 2], rs1_send.at[s],
                        rs1_recv.at[s], left)
            r0.start()
            r1.start()
            c0 = (my - s - 1) % N_DEV
            c1 = (my + s + 1) % N_DEV
            cx0 = pltpu.make_async_copy(
                x_hbm.at[pl.ds(c0 * CH, CH), :], xs0, lx0)
            cx1 = pltpu.make_async_copy(
                x_hbm.at[pl.ds(c1 * CH, CH), :], xs1, lx1)
            cx0.start()
            cx1.start()
            cx0.wait()
            cx1.wait()
            dq0[...] = jnp.dot(xs0[...], w_ref[:, pl.ds(0, H)],
                               preferred_element_type=jnp.float32)
            dq1[...] = jnp.dot(xs1[...], w_ref[:, pl.ds(H, H)],
                               preferred_element_type=jnp.float32)
            r0.wait()
            r1.wait()
            acc0[...] = dq0[...] + comm0[s % 2].astype(jnp.float32)
            acc1[...] = dq1[...] + comm1[s % 2].astype(jnp.float32)

        my_amax = jnp.maximum(jnp.max(jnp.abs(acc0[...])),
                              jnp.max(jnp.abs(acc1[...])))
        axb[0] = jnp.full((8, 128), my_amax, jnp.float32)
        a_r = remote(axb.at[0], axb.at[1], ax_send.at[0], ax_recv.at[0],
                     right)
        a_l = remote(axb.at[0], axb.at[2], ax_send.at[1], ax_recv.at[1],
                     left)
        a_r.start()
        a_l.start()
        a_r.wait()
        a_l.wait()
        a_f = remote(axb.at[1], axb.at[3], ax_send.at[2], ax_recv.at[2],
                     right)
        a_f.start()
        a_f.wait()
        amax = jnp.max(jnp.stack([axb[0], axb[1], axb[2], axb[3]]))
        scale = amax / 127.0
        inv = 127.0 / amax

        g0 = (my + 1) % N_DEV
        g1 = (my - 1) % N_DEV
        q0[...] = jnp.clip(jnp.round(acc0[...] * inv), -127.0, 127.0
                           ).astype(jnp.int8)
        q1[...] = jnp.clip(jnp.round(acc1[...] * inv), -127.0, 127.0
                           ).astype(jnp.int8)
        dq0[...] = q0[...].astype(jnp.float32) * scale
        dq1[...] = q1[...].astype(jnp.float32) * scale
        st0 = pltpu.make_async_copy(
            dq0, out_hbm.at[pl.ds(g0 * CH, CH), pl.ds(0, H)], lo0)
        st1 = pltpu.make_async_copy(
            dq1, out_hbm.at[pl.ds(g1 * CH, CH), pl.ds(H, H)], lo1)
        st0.start()
        st1.start()

        for s in range(N_DEV - 1):
            src0 = q0 if s == 0 else ag0.at[s - 1]
            src1 = q1 if s == 0 else ag1.at[s - 1]
            r0 = remote(src0, ag0.at[s], ag0_send.at[s], ag0_recv.at[s],
                        right)
            r1 = remote(src1, ag1.at[s], ag1_send.at[s], ag1_recv.at[s],
                        left)
            r0.start()
            r1.start()
            r0.wait()
            r1.wait()
            st0.wait()
            st1.wait()
            a0 = (my - s) % N_DEV
            a1 = (my + s) % N_DEV
            dq0[...] = ag0[s].astype(jnp.float32) * scale
            dq1[...] = ag1[s].astype(jnp.float32) * scale
            st0 = pltpu.make_async_copy(
                dq0, out_hbm.at[pl.ds(a0 * CH, CH), pl.ds(0, H)], lo0)
            st1 = pltpu.make_async_copy(
                dq1, out_hbm.at[pl.ds(a1 * CH, CH), pl.ds(H, H)], lo1)
            st0.start()
            st1.start()
        st0.wait()
        st1.wait()

    return pl.pallas_call(
        body,
        out_shape=jax.ShapeDtypeStruct((M, N), jnp.float32),
        in_specs=[
            pl.BlockSpec(memory_space=pl.ANY),
            pl.BlockSpec(memory_space=pltpu.MemorySpace.VMEM),
        ],
        out_specs=pl.BlockSpec(memory_space=pl.ANY),
        scratch_shapes=[
            pltpu.VMEM((CH, K // N_DEV), jnp.float32),
            pltpu.VMEM((CH, K // N_DEV), jnp.float32),
            pltpu.VMEM((CH, H), jnp.float32),
            pltpu.VMEM((CH, H), jnp.float32),
            pltpu.VMEM((CH, H), jnp.float32),
            pltpu.VMEM((CH, H), jnp.float32),
            pltpu.VMEM((CH, H), jnp.bfloat16),
            pltpu.VMEM((CH, H), jnp.bfloat16),
            pltpu.VMEM((2, CH, H), jnp.bfloat16),
            pltpu.VMEM((2, CH, H), jnp.bfloat16),
            pltpu.VMEM((CH, H), jnp.int8),
            pltpu.VMEM((CH, H), jnp.int8),
            pltpu.VMEM((N_DEV - 1, CH, H), jnp.int8),
            pltpu.VMEM((N_DEV - 1, CH, H), jnp.int8),
            pltpu.VMEM((4, 8, 128), jnp.float32),
            pltpu.SemaphoreType.DMA((N_DEV - 1,)),
            pltpu.SemaphoreType.DMA((N_DEV - 1,)),
            pltpu.SemaphoreType.DMA((N_DEV - 1,)),
            pltpu.SemaphoreType.DMA((N_DEV - 1,)),
            pltpu.SemaphoreType.DMA((N_DEV - 1,)),
            pltpu.SemaphoreType.DMA((N_DEV - 1,)),
            pltpu.SemaphoreType.DMA((N_DEV - 1,)),
            pltpu.SemaphoreType.DMA((N_DEV - 1,)),
            pltpu.SemaphoreType.DMA((3,)),
            pltpu.SemaphoreType.DMA((3,)),
            pltpu.SemaphoreType.DMA,
            pltpu.SemaphoreType.DMA,
            pltpu.SemaphoreType.DMA,
            pltpu.SemaphoreType.DMA,
        ],
        compiler_params=pltpu.CompilerParams(
            collective_id=0,
            vmem_limit_bytes=63 * 1024 * 1024,
        ),
    )(x, w_mat)


def kernel(x, w_mat):
    return _fused_gemm_ar_quant(x, w_mat)


# device time: 148086 ns/iter; 4.9266x vs baseline; 1.0992x over previous
import jax
import jax.numpy as jnp
from jax import lax
from jax.experimental import pallas as pl
from jax.experimental.pallas import tpu as pltpu

N_DEV = 4
M, K, N = 4096, 4096, 2048
CH = M // N_DEV
HH = CH // 2
H = N // 2
NS = N_DEV - 1


def _fused_gemm_ar_quant(x, w_mat):
    kx = x.shape[1]

    def body(x_hbm, w_ref, out_hbm,
             xs0, xs1, acc0, acc1, dq0, dq1, sb0, sb1, comm0, comm1,
             q0, q1, ag0, ag1, axb,
             rs0_send, rs0_recv, rs1_send, rs1_recv,
             ag0_send, ag0_recv, ag1_send, ag1_recv,
             ax_send, ax_recv, lx0, lx1, lo0, lo1):
        my = lax.axis_index("i")
        left = (my - 1) % N_DEV
        right = (my + 1) % N_DEV

        def remote(src, dst, ssem, rsem, tgt):
            return pltpu.make_async_remote_copy(
                src_ref=src, dst_ref=dst, send_sem=ssem, recv_sem=rsem,
                device_id=(tgt,), device_id_type=pl.DeviceIdType.MESH,
            )

        barrier = pltpu.get_barrier_semaphore()
        for nbr in (left, right):
            pl.semaphore_signal(
                barrier, inc=1,
                device_id=(nbr,), device_id_type=pl.DeviceIdType.MESH,
            )
        pl.semaphore_wait(barrier, 2)

        cx = pltpu.make_async_copy(
            x_hbm.at[pl.ds(my * CH, CH), :], xs0, lx0)
        cx.start()
        cx.wait()
        r0_cur = [None, None]
        r1_cur = [None, None]
        for h in range(2):
            rh = pl.ds(h * HH, HH)
            acc0[rh, :] = jnp.dot(xs0[rh, :], w_ref[:, pl.ds(0, H)],
                                  preferred_element_type=jnp.float32)
            acc1[rh, :] = jnp.dot(xs0[rh, :], w_ref[:, pl.ds(H, H)],
                                  preferred_element_type=jnp.float32)
            sb0[rh, :] = acc0[rh, :].astype(jnp.bfloat16)
            sb1[rh, :] = acc1[rh, :].astype(jnp.bfloat16)
            crow = pl.ds(h * HH, HH)
            r0_cur[h] = remote(sb0.at[rh, :], comm0.at[crow, :],
                               rs0_send.at[h], rs0_recv.at[h], right)
            r1_cur[h] = remote(sb1.at[rh, :], comm1.at[crow, :],
                               rs1_send.at[h], rs1_recv.at[h], left)
            r0_cur[h].start()
            r1_cur[h].start()

        for s in range(NS):
            c0 = (my - s - 1) % N_DEV
            c1 = (my + s + 1) % N_DEV
            cx0 = pltpu.make_async_copy(
                x_hbm.at[pl.ds(c0 * CH, CH), :], xs0, lx0)
            cx1 = pltpu.make_async_copy(
                x_hbm.at[pl.ds(c1 * CH, CH), :], xs1, lx1)
            cx0.start()
            cx1.start()
            cx0.wait()
            cx1.wait()
            dq0[...] = jnp.dot(xs0[...], w_ref[:, pl.ds(0, H)],
                               preferred_element_type=jnp.float32)
            dq1[...] = jnp.dot(xs1[...], w_ref[:, pl.ds(H, H)],
                               preferred_element_type=jnp.float32)
            r0_nxt = [None, None]
            r1_nxt = [None, None]
            for h in range(2):
                rh = pl.ds(h * HH, HH)
                r0_cur[h].wait()
                r1_cur[h].wait()
                crow = pl.ds(s * CH + h * HH, HH)
                acc0[rh, :] = dq0[rh, :] + comm0[crow, :].astype(jnp.float32)
                acc1[rh, :] = dq1[rh, :] + comm1[crow, :].astype(jnp.float32)
                if s < NS - 1:
                    sb0[rh, :] = acc0[rh, :].astype(jnp.bfloat16)
                    sb1[rh, :] = acc1[rh, :].astype(jnp.bfloat16)
                    nrow = pl.ds((s + 1) * CH + h * HH, HH)
                    i = 2 * (s + 1) + h
                    r0_nxt[h] = remote(sb0.at[rh, :], comm0.at[nrow, :],
                                       rs0_send.at[i], rs0_recv.at[i],
                                       right)
                    r1_nxt[h] = remote(sb1.at[rh, :], comm1.at[nrow, :],
                                       rs1_send.at[i], rs1_recv.at[i],
                                       left)
                    r0_nxt[h].start()
                    r1_nxt[h].start()
            r0_cur, r1_cur = r0_nxt, r1_nxt

        my_amax = jnp.maximum(jnp.max(jnp.abs(acc0[...])),
                              jnp.max(jnp.abs(acc1[...])))
        axb[0] = jnp.full((8, 128), my_amax, jnp.float32)
        a_r = remote(axb.at[0], axb.at[1], ax_send.at[0], ax_recv.at[0],
                     right)
        a_l = remote(axb.at[0], axb.at[2], ax_send.at[1], ax_recv.at[1],
                     left)
        a_r.start()
        a_l.start()
        a_r.wait()
        a_l.wait()
        a_f = remote(axb.at[1], axb.at[3], ax_send.at[2], ax_recv.at[2],
                     right)
        a_f.start()
        a_f.wait()
        amax = jnp.max(jnp.stack([axb[0], axb[1], axb[2], axb[3]]))
        scale = amax / 127.0
        inv = 127.0 / amax

        g0 = (my + 1) % N_DEV
        g1 = (my - 1) % N_DEV
        ag0_cur = [None, None]
        ag1_cur = [None, None]
        for h in range(2):
            rh = pl.ds(h * HH, HH)
            q0[rh, :] = jnp.clip(jnp.round(acc0[rh, :] * inv),
                                 -127.0, 127.0).astype(jnp.int8)
            q1[rh, :] = jnp.clip(jnp.round(acc1[rh, :] * inv),
                                 -127.0, 127.0).astype(jnp.int8)
            crow = pl.ds(h * HH, HH)
            ag0_cur[h] = remote(q0.at[rh, :], ag0.at[crow, :],
                                ag0_send.at[h], ag0_recv.at[h], right)
            ag1_cur[h] = remote(q1.at[rh, :], ag1.at[crow, :],
                                ag1_send.at[h], ag1_recv.at[h], left)
            ag0_cur[h].start()
            ag1_cur[h].start()
        dq0[...] = q0[...].astype(jnp.float32) * scale
        dq1[...] = q1[...].astype(jnp.float32) * scale
        st0 = pltpu.make_async_copy(
            dq0, out_hbm.at[pl.ds(g0 * CH, CH), pl.ds(0, H)], lo0)
        st1 = pltpu.make_async_copy(
            dq1, out_hbm.at[pl.ds(g1 * CH, CH), pl.ds(H, H)], lo1)
        st0.start()
        st1.start()

        for s in range(NS):
            ag0_nxt = [None, None]
            ag1_nxt = [None, None]
            for h in range(2):
                ag0_cur[h].wait()
                ag1_cur[h].wait()
                if s < NS - 1:
                    srow = pl.ds(s * CH + h * HH, HH)
                    nrow = pl.ds((s + 1) * CH + h * HH, HH)
                    i = 2 * (s + 1) + h
                    ag0_nxt[h] = remote(ag0.at[srow, :], ag0.at[nrow, :],
                                        ag0_send.at[i], ag0_recv.at[i],
                                        right)
                    ag1_nxt[h] = remote(ag1.at[srow, :], ag1.at[nrow, :],
                                        ag1_send.at[i], ag1_recv.at[i],
                                        left)
                    ag0_nxt[h].start()
                    ag1_nxt[h].start()
            st0.wait()
            st1.wait()
            a0 = (my - s) % N_DEV
            a1 = (my + s) % N_DEV
            srow = pl.ds(s * CH, CH)
            dq0[...] = ag0[srow, :].astype(jnp.float32) * scale
            dq1[...] = ag1[srow, :].astype(jnp.float32) * scale
            st0 = pltpu.make_async_copy(
                dq0, out_hbm.at[pl.ds(a0 * CH, CH), pl.ds(0, H)], lo0)
            st1 = pltpu.make_async_copy(
                dq1, out_hbm.at[pl.ds(a1 * CH, CH), pl.ds(H, H)], lo1)
            st0.start()
            st1.start()
            ag0_cur, ag1_cur = ag0_nxt, ag1_nxt
        st0.wait()
        st1.wait()

    return pl.pallas_call(
        body,
        out_shape=jax.ShapeDtypeStruct((M, N), jnp.float32),
        in_specs=[
            pl.BlockSpec(memory_space=pl.ANY),
            pl.BlockSpec(memory_space=pltpu.MemorySpace.VMEM),
        ],
        out_specs=pl.BlockSpec(memory_space=pl.ANY),
        scratch_shapes=[
            pltpu.VMEM((CH, kx), jnp.float32),
            pltpu.VMEM((CH, kx), jnp.float32),
            pltpu.VMEM((CH, H), jnp.float32),
            pltpu.VMEM((CH, H), jnp.float32),
            pltpu.VMEM((CH, H), jnp.float32),
            pltpu.VMEM((CH, H), jnp.float32),
            pltpu.VMEM((CH, H), jnp.bfloat16),
            pltpu.VMEM((CH, H), jnp.bfloat16),
            pltpu.VMEM((NS * CH, H), jnp.bfloat16),
            pltpu.VMEM((NS * CH, H), jnp.bfloat16),
            pltpu.VMEM((CH, H), jnp.int8),
            pltpu.VMEM((CH, H), jnp.int8),
            pltpu.VMEM((NS * CH, H), jnp.int8),
            pltpu.VMEM((NS * CH, H), jnp.int8),
            pltpu.VMEM((4, 8, 128), jnp.float32),
            pltpu.SemaphoreType.DMA((2 * NS,)),
            pltpu.SemaphoreType.DMA((2 * NS,)),
            pltpu.SemaphoreType.DMA((2 * NS,)),
            pltpu.SemaphoreType.DMA((2 * NS,)),
            pltpu.SemaphoreType.DMA((2 * NS,)),
            pltpu.SemaphoreType.DMA((2 * NS,)),
            pltpu.SemaphoreType.DMA((2 * NS,)),
            pltpu.SemaphoreType.DMA((2 * NS,)),
            pltpu.SemaphoreType.DMA((3,)),
            pltpu.SemaphoreType.DMA((3,)),
            pltpu.SemaphoreType.DMA,
            pltpu.SemaphoreType.DMA,
            pltpu.SemaphoreType.DMA,
            pltpu.SemaphoreType.DMA,
        ],
        compiler_params=pltpu.CompilerParams(
            collective_id=0,
            vmem_limit_bytes=63 * 1024 * 1024,
        ),
    )(x, w_mat)


def kernel(x, w_mat):
    return _fused_gemm_ar_quant(x, w_mat)


# device time: 148001 ns/iter; 4.9295x vs baseline; 1.0006x over previous
import jax
import jax.numpy as jnp
from jax import lax
from jax.experimental import pallas as pl
from jax.experimental.pallas import tpu as pltpu

N_DEV = 4
M, K, N = 4096, 4096, 2048
CH = M // N_DEV
HH = CH // 2
H = N // 2
NS = N_DEV - 1


def _fused_gemm_ar_quant(x, w_mat):
    kx = x.shape[1]

    def body(x_hbm, w_ref, out_hbm,
             xs0, xs1, acc0, acc1, dq0, dq1, sb0, sb1, comm0, comm1,
             q0, q1, ag0, ag1, axb,
             rs0_send, rs0_recv, rs1_send, rs1_recv,
             ag0_send, ag0_recv, ag1_send, ag1_recv,
             ax_send, ax_recv, lx0, lx1, lo0, lo1):
        my = lax.axis_index("i")
        left = (my - 1) % N_DEV
        right = (my + 1) % N_DEV

        def remote(src, dst, ssem, rsem, tgt):
            return pltpu.make_async_remote_copy(
                src_ref=src, dst_ref=dst, send_sem=ssem, recv_sem=rsem,
                device_id=(tgt,), device_id_type=pl.DeviceIdType.MESH,
            )

        cx_my = pltpu.make_async_copy(
            x_hbm.at[pl.ds(my * CH, CH), :], xs1, lx1)
        cx_c0 = pltpu.make_async_copy(
            x_hbm.at[pl.ds(((my - 1) % N_DEV) * CH, CH), :], xs0, lx0)
        cx_my.start()
        cx_c0.start()

        barrier = pltpu.get_barrier_semaphore()
        for nbr in (left, right):
            pl.semaphore_signal(
                barrier, inc=1,
                device_id=(nbr,), device_id_type=pl.DeviceIdType.MESH,
            )
        pl.semaphore_wait(barrier, 2)

        cx_my.wait()
        r0_cur = [None, None]
        r1_cur = [None, None]
        for h in range(2):
            rh = pl.ds(h * HH, HH)
            crow = pl.ds(h * HH, HH)
            acc0[rh, :] = jnp.dot(xs1[rh, :], w_ref[:, pl.ds(0, H)],
                                  preferred_element_type=jnp.float32)
            sb0[rh, :] = acc0[rh, :].astype(jnp.bfloat16)
            r0_cur[h] = remote(sb0.at[rh, :], comm0.at[crow, :],
                               rs0_send.at[h], rs0_recv.at[h], right)
            r0_cur[h].start()
            acc1[rh, :] = jnp.dot(xs1[rh, :], w_ref[:, pl.ds(H, H)],
                                  preferred_element_type=jnp.float32)
            sb1[rh, :] = acc1[rh, :].astype(jnp.bfloat16)
            r1_cur[h] = remote(sb1.at[rh, :], comm1.at[crow, :],
                               rs1_send.at[h], rs1_recv.at[h], left)
            r1_cur[h].start()

        for s in range(NS):
            c0 = (my - s - 1) % N_DEV
            c1 = (my + s + 1) % N_DEV
            cx1 = pltpu.make_async_copy(
                x_hbm.at[pl.ds(c1 * CH, CH), :], xs1, lx1)
            cx1.start()
            if s == 0:
                cx_c0.wait()
            else:
                cx0 = pltpu.make_async_copy(
                    x_hbm.at[pl.ds(c0 * CH, CH), :], xs0, lx0)
                cx0.start()
                cx0.wait()
            cx1.wait()
            dq0[...] = jnp.dot(xs0[...], w_ref[:, pl.ds(0, H)],
                               preferred_element_type=jnp.float32)
            dq1[...] = jnp.dot(xs1[...], w_ref[:, pl.ds(H, H)],
                               preferred_element_type=jnp.float32)
            r0_nxt = [None, None]
            r1_nxt = [None, None]
            for h in range(2):
                rh = pl.ds(h * HH, HH)
                r0_cur[h].wait()
                r1_cur[h].wait()
                crow = pl.ds(s * CH + h * HH, HH)
                acc0[rh, :] = dq0[rh, :] + comm0[crow, :].astype(jnp.float32)
                acc1[rh, :] = dq1[rh, :] + comm1[crow, :].astype(jnp.float32)
                if s < NS - 1:
                    sb0[rh, :] = acc0[rh, :].astype(jnp.bfloat16)
                    sb1[rh, :] = acc1[rh, :].astype(jnp.bfloat16)
                    nrow = pl.ds((s + 1) * CH + h * HH, HH)
                    i = 2 * (s + 1) + h
                    r0_nxt[h] = remote(sb0.at[rh, :], comm0.at[nrow, :],
                                       rs0_send.at[i], rs0_recv.at[i],
                                       right)
                    r1_nxt[h] = remote(sb1.at[rh, :], comm1.at[nrow, :],
                                       rs1_send.at[i], rs1_recv.at[i],
                                       left)
                    r0_nxt[h].start()
                    r1_nxt[h].start()
            r0_cur, r1_cur = r0_nxt, r1_nxt

        my_amax = jnp.maximum(jnp.max(jnp.abs(acc0[...])),
                              jnp.max(jnp.abs(acc1[...])))
        axb[0] = jnp.full((8, 128), my_amax, jnp.float32)
        a_r = remote(axb.at[0], axb.at[1], ax_send.at[0], ax_recv.at[0],
                     right)
        a_l = remote(axb.at[0], axb.at[2], ax_send.at[1], ax_recv.at[1],
                     left)
        a_r.start()
        a_l.start()
        a_r.wait()
        a_l.wait()
        a_f = remote(axb.at[1], axb.at[3], ax_send.at[2], ax_recv.at[2],
                     right)
        a_f.start()
        a_f.wait()
        amax = jnp.max(jnp.stack([axb[0], axb[1], axb[2], axb[3]]))
        scale = amax / 127.0
        inv = 127.0 / amax

        g0 = (my + 1) % N_DEV
        g1 = (my - 1) % N_DEV
        ag0_cur = [None, None]
        ag1_cur = [None, None]
        for h in range(2):
            rh = pl.ds(h * HH, HH)
            q0[rh, :] = jnp.clip(jnp.round(acc0[rh, :] * inv),
                                 -127.0, 127.0).astype(jnp.int8)
            q1[rh, :] = jnp.clip(jnp.round(acc1[rh, :] * inv),
                                 -127.0, 127.0).astype(jnp.int8)
            crow = pl.ds(h * HH, HH)
            ag0_cur[h] = remote(q0.at[rh, :], ag0.at[crow, :],
                                ag0_send.at[h], ag0_recv.at[h], right)
            ag1_cur[h] = remote(q1.at[rh, :], ag1.at[crow, :],
                                ag1_send.at[h], ag1_recv.at[h], left)
            ag0_cur[h].start()
            ag1_cur[h].start()
        dq0[...] = q0[...].astype(jnp.float32) * scale
        dq1[...] = q1[...].astype(jnp.float32) * scale
        st0 = pltpu.make_async_copy(
            dq0, out_hbm.at[pl.ds(g0 * CH, CH), pl.ds(0, H)], lo0)
        st1 = pltpu.make_async_copy(
            dq1, out_hbm.at[pl.ds(g1 * CH, CH), pl.ds(H, H)], lo1)
        st0.start()
        st1.start()

        for s in range(NS):
            ag0_nxt = [None, None]
            ag1_nxt = [None, None]
            for h in range(2):
                ag0_cur[h].wait()
                ag1_cur[h].wait()
                if s < NS - 1:
                    srow = pl.ds(s * CH + h * HH, HH)
                    nrow = pl.ds((s + 1) * CH + h * HH, HH)
                    i = 2 * (s + 1) + h
                    ag0_nxt[h] = remote(ag0.at[srow, :], ag0.at[nrow, :],
                                        ag0_send.at[i], ag0_recv.at[i],
                                        right)
                    ag1_nxt[h] = remote(ag1.at[srow, :], ag1.at[nrow, :],
                                        ag1_send.at[i], ag1_recv.at[i],
                                        left)
                    ag0_nxt[h].start()
                    ag1_nxt[h].start()
            st0.wait()
            st1.wait()
            a0 = (my - s) % N_DEV
            a1 = (my + s) % N_DEV
            srow = pl.ds(s * CH, CH)
            dq0[...] = ag0[srow, :].astype(jnp.float32) * scale
            dq1[...] = ag1[srow, :].astype(jnp.float32) * scale
            st0 = pltpu.make_async_copy(
                dq0, out_hbm.at[pl.ds(a0 * CH, CH), pl.ds(0, H)], lo0)
            st1 = pltpu.make_async_copy(
                dq1, out_hbm.at[pl.ds(a1 * CH, CH), pl.ds(H, H)], lo1)
            st0.start()
            st1.start()
            ag0_cur, ag1_cur = ag0_nxt, ag1_nxt
        st0.wait()
        st1.wait()

    return pl.pallas_call(
        body,
        out_shape=jax.ShapeDtypeStruct((M, N), jnp.float32),
        in_specs=[
            pl.BlockSpec(memory_space=pl.ANY),
            pl.BlockSpec(memory_space=pltpu.MemorySpace.VMEM),
        ],
        out_specs=pl.BlockSpec(memory_space=pl.ANY),
        scratch_shapes=[
            pltpu.VMEM((CH, kx), jnp.float32),
            pltpu.VMEM((CH, kx), jnp.float32),
            pltpu.VMEM((CH, H), jnp.float32),
            pltpu.VMEM((CH, H), jnp.float32),
            pltpu.VMEM((CH, H), jnp.float32),
            pltpu.VMEM((CH, H), jnp.float32),
            pltpu.VMEM((CH, H), jnp.bfloat16),
            pltpu.VMEM((CH, H), jnp.bfloat16),
            pltpu.VMEM((NS * CH, H), jnp.bfloat16),
            pltpu.VMEM((NS * CH, H), jnp.bfloat16),
            pltpu.VMEM((CH, H), jnp.int8),
            pltpu.VMEM((CH, H), jnp.int8),
            pltpu.VMEM((NS * CH, H), jnp.int8),
            pltpu.VMEM((NS * CH, H), jnp.int8),
            pltpu.VMEM((4, 8, 128), jnp.float32),
            pltpu.SemaphoreType.DMA((2 * NS,)),
            pltpu.SemaphoreType.DMA((2 * NS,)),
            pltpu.SemaphoreType.DMA((2 * NS,)),
            pltpu.SemaphoreType.DMA((2 * NS,)),
            pltpu.SemaphoreType.DMA((2 * NS,)),
            pltpu.SemaphoreType.DMA((2 * NS,)),
            pltpu.SemaphoreType.DMA((2 * NS,)),
            pltpu.SemaphoreType.DMA((2 * NS,)),
            pltpu.SemaphoreType.DMA((3,)),
            pltpu.SemaphoreType.DMA((3,)),
            pltpu.SemaphoreType.DMA,
            pltpu.SemaphoreType.DMA,
            pltpu.SemaphoreType.DMA,
            pltpu.SemaphoreType.DMA,
        ],
        compiler_params=pltpu.CompilerParams(
            collective_id=0,
            vmem_limit_bytes=63 * 1024 * 1024,
        ),
    )(x, w_mat)


def kernel(x, w_mat):
    return _fused_gemm_ar_quant(x, w_mat)
